# baseline (speedup 1.0000x reference)
import functools

import jax
import jax.numpy as jnp
from jax.experimental import pallas as pl
from jax.experimental.pallas import tpu as pltpu
from jax.experimental.pallas import tpu_sc as plsc

_BATCH = 16384
_SEQ = 30
_EMB = 64
_HIDD = 50
_CLASS = 128
_NUM_EMB = 1000000
_N_IDX = _BATCH * _SEQ
_W = 256
_WI = 128

_R = 4096
_NG = (_NUM_EMB + 4 * _R - 1) // (4 * _R)
_LASTBLK = (_NUM_EMB - 1) // _R
_VROWS = _NG * _R

_NW = 32
_B_PER_W = _N_IDX // _NW
_GCHUNK = 512
_N_GCH = _B_PER_W // _GCHUNK

_CHUNK = 512


def _proj_body(t0_ref, t1_ref, t2_ref, t3_ref, w1p_ref, b1p_ref, o_ref):
    w1p = w1p_ref[...]
    b1p = b1p_ref[...]
    for q, t_ref in enumerate((t0_ref, t1_ref, t2_ref, t3_ref)):
        tq = t_ref[...].astype(jnp.bfloat16)
        hq = jnp.tanh(jax.lax.dot_general(
            tq, w1p, dimension_numbers=(((0,), (0,)), ((), ())),
            preferred_element_type=jnp.float32) + b1p)
        o_ref[:, q * _EMB:(q + 1) * _EMB] = hq.astype(jnp.bfloat16)


def _tc_project(tableT, w1p, b1p):
    tspec = lambda k: pl.BlockSpec(
        (_EMB, _R), lambda i, k=k: (0, jnp.minimum(4 * i + k, _LASTBLK)))
    return pl.pallas_call(
        _proj_body,
        grid=(_NG,),
        in_specs=[
            tspec(0), tspec(1), tspec(2), tspec(3),
            pl.BlockSpec((_EMB, _EMB), lambda i: (0, 0)),
            pl.BlockSpec((1, _EMB), lambda i: (0, 0)),
        ],
        out_specs=pl.BlockSpec((_R, _W), lambda i: (i, 0)),
        out_shape=jax.ShapeDtypeStruct((_VROWS, _W), jnp.bfloat16),
    )(tableT, tableT, tableT, tableT, w1p, b1p)


def _sc_gather(h_table, idx):
    mesh = plsc.VectorSubcoreMesh(core_axis_name="c", subcore_axis_name="s")

    @functools.partial(
        pl.kernel,
        out_type=jax.ShapeDtypeStruct((_N_IDX, _WI), jnp.int32),
        mesh=mesh,
        scratch_types=[
            pltpu.VMEM((_GCHUNK,), jnp.int32),
            pltpu.VMEM((_GCHUNK, _WI), jnp.int32),
            pltpu.SemaphoreType.DMA,
        ],
    )
    def gather_kernel(table_hbm, idx_hbm, out_hbm, idx_v, rows_v, sem):
        wid = jax.lax.axis_index("s") * 2 + jax.lax.axis_index("c")
        base = wid * _B_PER_W

        @pl.loop(0, _N_GCH)
        def _(j):
            off = base + j * _GCHUNK
            pltpu.sync_copy(idx_hbm.at[pl.ds(off, _GCHUNK)], idx_v)
            pltpu.async_copy(table_hbm.at[idx_v], rows_v, sem).wait()
            pltpu.sync_copy(rows_v, out_hbm.at[pl.ds(off, _GCHUNK)])

    return gather_kernel(h_table, idx)


def _mlp_body(e_ref, m_ref, w2a_ref, b2a_ref, w2b_ref, b2b_ref, w2c_ref,
              b2c_ref, o_ref):
    rows = e_ref.shape[1]
    lane_q = jax.lax.broadcasted_iota(jnp.int32, (rows, _W), 1) // _EMB
    acc = jnp.zeros((rows, _CLASS), jnp.float32)
    for s in range(_SEQ):
        es = e_ref[s]
        tok_q = m_ref[:, s:s + 1]
        keep = lane_q == tok_q
        es_sel = jnp.where(keep, es, jnp.bfloat16(0))
        acc = acc + jnp.dot(es_sel, w2a_ref[s],
                            preferred_element_type=jnp.float32)
    h2 = jnp.tanh(acc + b2a_ref[...]).astype(jnp.bfloat16)
    h3 = jnp.tanh(
        jnp.dot(h2, w2b_ref[...], preferred_element_type=jnp.float32)
        + b2b_ref[...]).astype(jnp.bfloat16)
    o_ref[...] = (
        jnp.dot(h3, w2c_ref[...], preferred_element_type=jnp.float32)
        + b2c_ref[...])


def _tc_mlp(e3, m, w2ad, b2a, w2b, b2b, w2c, b2c):
    return pl.pallas_call(
        _mlp_body,
        grid=(_BATCH // _CHUNK,),
        in_specs=[
            pl.BlockSpec((_SEQ, _CHUNK, _W), lambda i: (0, i, 0)),
            pl.BlockSpec((_CHUNK, _SEQ), lambda i: (i, 0)),
            pl.BlockSpec((_SEQ, _W, _CLASS), lambda i: (0, 0, 0)),
            pl.BlockSpec((1, _CLASS), lambda i: (0, 0)),
            pl.BlockSpec((_CLASS, 64), lambda i: (0, 0)),
            pl.BlockSpec((1, 64), lambda i: (0, 0)),
            pl.BlockSpec((64, _CLASS), lambda i: (0, 0)),
            pl.BlockSpec((1, _CLASS), lambda i: (0, 0)),
        ],
        out_specs=pl.BlockSpec((_CHUNK, _CLASS), lambda i: (i, 0)),
        out_shape=jax.ShapeDtypeStruct((_BATCH, _CLASS), jnp.float32),
    )(e3, m, w2ad, b2a, w2b, b2b, w2c, b2c)


def kernel(x, table, W1, b1, W2a, b2a, W2b, b2b, W2c, b2c):
    w1p = jnp.zeros((_EMB, _EMB), jnp.float32).at[:, :_HIDD].set(W1)
    w1p = w1p.astype(jnp.bfloat16)
    b1p = jnp.zeros((1, _EMB), jnp.float32).at[0, :_HIDD].set(b1)
    h_table = _tc_project(table.T, w1p, b1p)
    h_table_i = jax.lax.bitcast_convert_type(
        h_table.reshape(_VROWS, _WI, 2), jnp.int32)

    row = jnp.bitwise_or(
        jax.lax.shift_left(jax.lax.shift_right_logical(x, 14), 12),
        jnp.bitwise_and(x, 4095))
    m = jnp.bitwise_and(jax.lax.shift_right_logical(x, 12), 3)
    idx_t = row.T.reshape(_N_IDX)
    e = _sc_gather(h_table_i, idx_t)
    e3 = jax.lax.bitcast_convert_type(e, jnp.bfloat16).reshape(
        _SEQ, _BATCH, _W)

    w2a3 = W2a.reshape(_SEQ, _HIDD, _CLASS)
    w2ad = jnp.zeros((_SEQ, _W, _CLASS), jnp.float32)
    for q in range(4):
        w2ad = w2ad.at[:, q * _EMB:q * _EMB + _HIDD, :].set(w2a3)
    w2ad = w2ad.astype(jnp.bfloat16)
    return _tc_mlp(e3, m, w2ad, b2a.reshape(1, -1),
                   W2b.astype(jnp.bfloat16), b2b.reshape(1, -1),
                   W2c.astype(jnp.bfloat16), b2c.reshape(1, -1))


# speedup vs baseline: 6.6779x; 6.6779x over previous
import functools

import jax
import jax.numpy as jnp
from jax.experimental import pallas as pl
from jax.experimental.pallas import tpu as pltpu
from jax.experimental.pallas import tpu_sc as plsc

_BATCH = 16384
_SEQ = 30
_EMB = 64
_HIDD = 50
_CLASS = 128
_NUM_EMB = 1000000
_N_IDX = _BATCH * _SEQ
_W = 256
_WI = 128

_R = 4096
_NG = (_NUM_EMB + 4 * _R - 1) // (4 * _R)
_LASTBLK = (_NUM_EMB - 1) // _R
_VROWS = _NG * _R

_NW = 32
_B_PER_W = _N_IDX // _NW
_GCHUNK = 512
_N_GCH = _B_PER_W // _GCHUNK

_CHUNK = 512


def _proj_body(t0_ref, t1_ref, t2_ref, t3_ref, w1p_ref, b1p_ref, o_ref):
    w1p = w1p_ref[...]
    b1p = b1p_ref[...]
    hs = []
    for t_ref in (t0_ref, t1_ref, t2_ref, t3_ref):
        tq = t_ref[...].astype(jnp.bfloat16)
        hs.append(jnp.tanh(jax.lax.dot_general(
            tq, w1p, dimension_numbers=(((0,), (0,)), ((), ())),
            preferred_element_type=jnp.float32) + b1p))
    a = jax.lax.bitcast_convert_type(
        jnp.concatenate(hs[:2], axis=1), jnp.uint32)
    b = jax.lax.bitcast_convert_type(
        jnp.concatenate(hs[2:], axis=1), jnp.uint32)
    half = jnp.uint32(0x8000)
    lo = jax.lax.shift_right_logical(a + half, jnp.uint32(16))
    hi = jnp.bitwise_and(b + half, jnp.uint32(0xFFFF0000))
    o_ref[...] = jax.lax.bitcast_convert_type(
        jnp.bitwise_or(lo, hi), jnp.int32)


def _tc_project(tableT, w1p, b1p):
    tspec = lambda k: pl.BlockSpec(
        (_EMB, _R), lambda i, k=k: (0, jnp.minimum(4 * i + k, _LASTBLK)))
    return pl.pallas_call(
        _proj_body,
        grid=(_NG,),
        in_specs=[
            tspec(0), tspec(1), tspec(2), tspec(3),
            pl.BlockSpec((_EMB, _EMB), lambda i: (0, 0)),
            pl.BlockSpec((1, _EMB), lambda i: (0, 0)),
        ],
        out_specs=pl.BlockSpec((_R, _WI), lambda i: (i, 0)),
        out_shape=jax.ShapeDtypeStruct((_VROWS, _WI), jnp.int32),
    )(tableT, tableT, tableT, tableT, w1p, b1p)


def _sc_gather(h_table, idx):
    mesh = plsc.VectorSubcoreMesh(core_axis_name="c", subcore_axis_name="s")

    @functools.partial(
        pl.kernel,
        out_type=jax.ShapeDtypeStruct((_N_IDX, _WI), jnp.int32),
        mesh=mesh,
        scratch_types=[
            pltpu.VMEM((_GCHUNK,), jnp.int32),
            pltpu.VMEM((_GCHUNK, _WI), jnp.int32),
            pltpu.SemaphoreType.DMA,
        ],
    )
    def gather_kernel(table_hbm, idx_hbm, out_hbm, idx_v, rows_v, sem):
        wid = jax.lax.axis_index("s") * 2 + jax.lax.axis_index("c")
        base = wid * _B_PER_W

        @pl.loop(0, _N_GCH)
        def _(j):
            off = base + j * _GCHUNK
            pltpu.sync_copy(idx_hbm.at[pl.ds(off, _GCHUNK)], idx_v)
            pltpu.async_copy(table_hbm.at[idx_v], rows_v, sem).wait()
            pltpu.sync_copy(rows_v, out_hbm.at[pl.ds(off, _GCHUNK)])

    return gather_kernel(h_table, idx)


def _mlp_body(e_ref, m_ref, w2a_ref, b2a_ref, w2b_ref, b2b_ref, w2c_ref,
              b2c_ref, o_ref):
    rows = e_ref.shape[1]
    lane_h = jax.lax.broadcasted_iota(jnp.int32, (rows, _WI), 1) // _EMB
    acc = jnp.zeros((rows, _CLASS), jnp.float32)
    for s in range(_SEQ):
        es = e_ref[s]
        tok_q = m_ref[:, s:s + 1]
        f_lo = jax.lax.bitcast_convert_type(
            jax.lax.shift_left(es, jnp.int32(16)), jnp.float32)
        f_hi = jax.lax.bitcast_convert_type(
            jnp.bitwise_and(es, jnp.int32(-65536)), jnp.float32)
        f = jnp.where(tok_q >= 2, f_hi, f_lo)
        keep = lane_h == jnp.bitwise_and(tok_q, 1)
        es_sel = jnp.where(keep, f, 0.0).astype(jnp.bfloat16)
        acc = acc + jnp.dot(es_sel, w2a_ref[s],
                            preferred_element_type=jnp.float32)
    h2 = jnp.tanh(acc + b2a_ref[...]).astype(jnp.bfloat16)
    h3 = jnp.tanh(
        jnp.dot(h2, w2b_ref[...], preferred_element_type=jnp.float32)
        + b2b_ref[...]).astype(jnp.bfloat16)
    o_ref[...] = (
        jnp.dot(h3, w2c_ref[...], preferred_element_type=jnp.float32)
        + b2c_ref[...])


def _tc_mlp(e3, m, w2ad, b2a, w2b, b2b, w2c, b2c):
    return pl.pallas_call(
        _mlp_body,
        grid=(_BATCH // _CHUNK,),
        in_specs=[
            pl.BlockSpec((_SEQ, _CHUNK, _WI), lambda i: (0, i, 0)),
            pl.BlockSpec((_CHUNK, _SEQ), lambda i: (i, 0)),
            pl.BlockSpec((_SEQ, _WI, _CLASS), lambda i: (0, 0, 0)),
            pl.BlockSpec((1, _CLASS), lambda i: (0, 0)),
            pl.BlockSpec((_CLASS, 64), lambda i: (0, 0)),
            pl.BlockSpec((1, 64), lambda i: (0, 0)),
            pl.BlockSpec((64, _CLASS), lambda i: (0, 0)),
            pl.BlockSpec((1, _CLASS), lambda i: (0, 0)),
        ],
        out_specs=pl.BlockSpec((_CHUNK, _CLASS), lambda i: (i, 0)),
        out_shape=jax.ShapeDtypeStruct((_BATCH, _CLASS), jnp.float32),
    )(e3, m, w2ad, b2a, w2b, b2b, w2c, b2c)


def kernel(x, table, W1, b1, W2a, b2a, W2b, b2b, W2c, b2c):
    w1p = jnp.zeros((_EMB, _EMB), jnp.float32).at[:, :_HIDD].set(W1)
    w1p = w1p.astype(jnp.bfloat16)
    b1p = jnp.zeros((1, _EMB), jnp.float32).at[0, :_HIDD].set(b1)
    h_table = _tc_project(table.T, w1p, b1p)

    row = jnp.bitwise_or(
        jax.lax.shift_left(jax.lax.shift_right_logical(x, 14), 12),
        jnp.bitwise_and(x, 4095))
    m = jnp.bitwise_and(jax.lax.shift_right_logical(x, 12), 3)
    idx_t = row.T.reshape(_N_IDX)
    e = _sc_gather(h_table, idx_t)
    e3 = e.reshape(_SEQ, _BATCH, _WI)

    w2a3 = W2a.reshape(_SEQ, _HIDD, _CLASS)
    w2ad = jnp.zeros((_SEQ, _WI, _CLASS), jnp.float32)
    w2ad = (w2ad.at[:, :_HIDD, :].set(w2a3)
            .at[:, _EMB:_EMB + _HIDD, :].set(w2a3)).astype(jnp.bfloat16)
    return _tc_mlp(e3, m, w2ad, b2a.reshape(1, -1),
                   W2b.astype(jnp.bfloat16), b2b.reshape(1, -1),
                   W2c.astype(jnp.bfloat16), b2c.reshape(1, -1))
